# baseline (speedup 1.0000x reference)
import functools

import jax
import jax.numpy as jnp
from jax import lax
from jax.experimental import pallas as pl
from jax.experimental.pallas import tpu as pltpu
from jax.experimental.pallas import tpu_sc as plsc

N = 10000
E = 320000
D = 128
R = 200

NC = 2
NS = 16
NW = NC * NS
K = 128
NCH = 79
EPW = NCH * K
EPADA = NW * EPW - E
NTRASH = 128
NPAIR = (NCH - 1) // 2
RSTRIDE = 624
RSPAN = 640

RH = R // NC
NLOC = N * RH
MC = 1015808
CSTR = MC // NS
CW = 2048
TRASH = NLOC
KC = 2048
NCC = 10
EPAD = NS * NCC * KC - E

_mesh = plsc.VectorSubcoreMesh(core_axis_name="c", subcore_axis_name="s")


def _agg_phase(x_hbm, src_hbm, dst_hbm, zero_hbm, out_hbm,
                 s0, s1, d0, d1, xb0, xb1, agg_sh,
                 semse0, semse1, semd0, semd1, semg0, semg1, sems0, sems1):
    c = lax.axis_index("c")
    s = lax.axis_index("s")
    wid = c * NS + s

    pltpu.sync_copy(zero_hbm.at[pl.ds(s * RSTRIDE, RSPAN)],
                    agg_sh.at[pl.ds(s * RSTRIDE, RSPAN)])
    plsc.subcore_barrier()

    sb = (s0, s1)
    db = (d0, d1)
    xb = (xb0, xb1)
    semse = (semse0, semse1)
    semd = (semd0, semd1)
    semg = (semg0, semg1)
    sems = (sems0, sems1)

    def fetch_src(i, p):
        pltpu.async_copy(src_hbm.at[wid, i], sb[p], semse[p])

    def fetch_dst(i, p):
        pltpu.async_copy(dst_hbm.at[wid, i], db[p], semd[p])

    def issue_gather(p):
        pltpu.async_copy(x_hbm.at[sb[p].at[0]], xb[p], semg[p])

    def issue_scatter(p):
        pltpu.async_copy(xb[p], agg_sh.at[db[p].at[0]], sems[p], add=True)

    def drain(sem, ref):
        pltpu.make_async_copy(x_hbm.at[pl.ds(0, K)], ref, sem).wait()

    def drain_idx(semv, p, bufs):
        pltpu.make_async_copy(src_hbm.at[0, 0], bufs[p], semv[p]).wait()

    fetch_src(0, 0)
    fetch_src(1, 1)
    fetch_dst(0, 0)
    drain_idx(semse, 0, sb)
    issue_gather(0)

    def pair(t, carry):
        a = 2 * t
        drain(semg[0], xb[0])
        fetch_src(a + 2, 0)
        drain_idx(semd, 0, db)
        issue_scatter(0)

        @pl.when(t > 0)
        def _():
            drain(sems[1], xb[1])
        fetch_dst(a + 1, 1)
        drain_idx(semse, 1, sb)
        issue_gather(1)

        drain(semg[1], xb[1])

        @pl.when(t < NPAIR - 1)
        def _():
            fetch_src(a + 3, 1)
        drain_idx(semd, 1, db)
        issue_scatter(1)
        drain(sems[0], xb[0])
        fetch_dst(a + 2, 0)
        drain_idx(semse, 0, sb)
        issue_gather(0)
        return carry

    lax.fori_loop(0, NPAIR, pair, 0)
    drain(semg[0], xb[0])
    drain(sems[1], xb[1])
    drain_idx(semd, 0, db)
    issue_scatter(0)
    drain(sems[0], xb[0])

    plsc.subcore_barrier()
    pltpu.sync_copy(agg_sh.at[pl.ds(s * RSTRIDE, RSPAN)],
                    out_hbm.at[pl.ds(c * N + s * RSTRIDE, RSPAN)])


_sc_agg = pl.kernel(
    _agg_phase,
    out_type=jax.ShapeDtypeStruct((2 * N, D), jnp.float32),
    mesh=_mesh,
    scratch_types=[
        pltpu.VMEM((1, K), jnp.int32),
        pltpu.VMEM((1, K), jnp.int32),
        pltpu.VMEM((1, K), jnp.int32),
        pltpu.VMEM((1, K), jnp.int32),
        pltpu.VMEM((K, D), jnp.float32),
        pltpu.VMEM((K, D), jnp.float32),
        pltpu.VMEM_SHARED((N + NTRASH, D), jnp.float32),
        pltpu.SemaphoreType.DMA,
        pltpu.SemaphoreType.DMA,
        pltpu.SemaphoreType.DMA,
        pltpu.SemaphoreType.DMA,
        pltpu.SemaphoreType.DMA,
        pltpu.SemaphoreType.DMA,
        pltpu.SemaphoreType.DMA,
        pltpu.SemaphoreType.DMA,
    ],
)


def _cnt_phase(dstc_hbm, etc_hbm, outc_hbm,
                 dstc_v, etc_v, fb, ones_v, stage_v, stage2_v, c_sh,
                 semsc, semz, semst):
    c = lax.axis_index("c")
    s = lax.axis_index("s")

    for l in range(CW // 16):
        stage_v[pl.ds(l * 16, 16)] = jnp.full((16,), 0.0, jnp.float32)

    def zbody(j, carry):
        pltpu.async_copy(stage_v, c_sh.at[pl.ds(s * CSTR + j * CW, CW)], semz)
        return carry

    lax.fori_loop(0, CSTR // CW, zbody, 0)
    pltpu.sync_copy(dstc_hbm.at[s], dstc_v)
    pltpu.sync_copy(etc_hbm.at[s], etc_v)
    for l in range(8):
        ones_v[0, pl.ds(l * 16, 16)] = jnp.full((16,), 1.0, jnp.float32)

    def zdrain(j, carry):
        pltpu.make_async_copy(outc_hbm.at[pl.ds(0, CW)], stage_v, semz).wait()
        return carry

    lax.fori_loop(0, CSTR // CW, zdrain, 0)
    plsc.subcore_barrier()

    lo = c * RH

    def cbody(i, carry):
        for r in range(16):
            for l in range(8):
                dv = dstc_v[i, r, pl.ds(l * 16, 16)]
                ev = etc_v[i, r, pl.ds(l * 16, 16)]
                el = ev - lo
                ok = (el >= 0) & (el < RH)
                f = jnp.where(ok, dv * RH + el, TRASH + (dv & 4095))
                fb[i, r, 0, pl.ds(l * 16, 16)] = f
        for r in range(16):
            pltpu.async_copy(ones_v.at[0], c_sh.at[fb.at[i, r, 0]],
                             semsc, add=True)
        return carry

    lax.fori_loop(0, NCC, cbody, 0)

    def dr(i, carry):
        pltpu.make_async_copy(dstc_hbm.at[0, 0], dstc_v.at[0], semsc).wait()
        return carry

    lax.fori_loop(0, NCC, dr, 0)
    plsc.subcore_barrier()

    def dpair(u, carry):
        j0 = 2 * u
        j1 = 2 * u + 1

        @pl.when(u > 0)
        def _():
            pltpu.make_async_copy(outc_hbm.at[pl.ds(0, CW)], stage_v,
                                  semst).wait()
            pltpu.make_async_copy(outc_hbm.at[pl.ds(0, CW)], stage2_v,
                                  semst).wait()
        pltpu.sync_copy(c_sh.at[pl.ds(s * CSTR + j0 * CW, CW)], stage_v)
        pltpu.async_copy(stage_v,
                         outc_hbm.at[pl.ds(c * MC + s * CSTR + j0 * CW, CW)],
                         semst)
        pltpu.sync_copy(c_sh.at[pl.ds(s * CSTR + j1 * CW, CW)], stage2_v)
        pltpu.async_copy(stage2_v,
                         outc_hbm.at[pl.ds(c * MC + s * CSTR + j1 * CW, CW)],
                         semst)
        return carry

    lax.fori_loop(0, (CSTR // CW) // 2, dpair, 0)
    pltpu.make_async_copy(outc_hbm.at[pl.ds(0, CW)], stage_v, semst).wait()
    pltpu.make_async_copy(outc_hbm.at[pl.ds(0, CW)], stage2_v, semst).wait()
    j_last = CSTR // CW - 1
    pltpu.sync_copy(c_sh.at[pl.ds(s * CSTR + j_last * CW, CW)], stage_v)
    pltpu.sync_copy(stage_v,
                    outc_hbm.at[pl.ds(c * MC + s * CSTR + j_last * CW, CW)])


_sc_cnt = pl.kernel(
    _cnt_phase,
    out_type=jax.ShapeDtypeStruct((2 * MC,), jnp.float32),
    mesh=_mesh,
    scratch_types=[
        pltpu.VMEM((NCC, 16, 128), jnp.int32),
        pltpu.VMEM((NCC, 16, 128), jnp.int32),
        pltpu.VMEM((NCC, 16, 1, 128), jnp.int32),
        pltpu.VMEM((1, 128), jnp.float32),
        pltpu.VMEM((CW,), jnp.float32),
        pltpu.VMEM((CW,), jnp.float32),
        pltpu.VMEM_SHARED((MC,), jnp.float32),
        pltpu.SemaphoreType.DMA,
        pltpu.SemaphoreType.DMA,
        pltpu.SemaphoreType.DMA,
    ],
)


def _dense_body(relu, a0_ref, a1_ref, x_ref, c0_ref, c1_ref, rel_ref, wi_ref,
                wo_ref, b_ref, o_ref):
    xb = x_ref[...]
    m = a0_ref[...] + a1_ref[...] + xb
    dn = (((1,), (1,)), ((), ()))
    dc = (((1,), (0,)), ((), ()))
    wi = wi_ref[...]
    relw = lax.dot_general(rel_ref[...], wi, dn,
                           preferred_element_type=jnp.float32)
    t = lax.dot_general(m, wi, dn, preferred_element_type=jnp.float32)
    t = t + lax.dot_general(xb, wo_ref[...], dn,
                            preferred_element_type=jnp.float32)
    t = t - lax.dot_general(c0_ref[...], relw[0:RH, :], dc,
                            preferred_element_type=jnp.float32)
    t = t - lax.dot_general(c1_ref[...], relw[RH:R, :], dc,
                            preferred_element_type=jnp.float32)
    t = t + (b_ref[...] - relw[0:1, :])
    o_ref[...] = jnp.maximum(t, 0.0) if relu else t


BLK = 1000
GRID = N // BLK


def _dense(agg2, c0, c1, x, rel, w_i, w_o, b, relu):
    return pl.pallas_call(
        functools.partial(_dense_body, relu),
        grid=(GRID,),
        in_specs=[
            pl.BlockSpec((BLK, D), lambda i: (i, 0)),
            pl.BlockSpec((BLK, D), lambda i: (i + GRID, 0)),
            pl.BlockSpec((BLK, D), lambda i: (i, 0)),
            pl.BlockSpec((BLK, RH), lambda i: (i, 0)),
            pl.BlockSpec((BLK, RH), lambda i: (i, 0)),
            pl.BlockSpec((R, D), lambda i: (0, 0)),
            pl.BlockSpec((D, D), lambda i: (0, 0)),
            pl.BlockSpec((D, D), lambda i: (0, 0)),
            pl.BlockSpec((1, D), lambda i: (0, 0)),
        ],
        out_specs=pl.BlockSpec((BLK, D), lambda i: (i, 0)),
        out_shape=jax.ShapeDtypeStruct((N, D), jnp.float32),
    )(agg2, agg2, x, c0, c1, rel, w_i, w_o, b)


def kernel(x, edge_index, edge_type, W_I1, W_O1, W_R1, rel1, b1,
           W_I2, W_O2, W_R2, rel2, b2):
    pad_i = jnp.arange(EPADA, dtype=jnp.int32)
    src = jnp.concatenate(
        [edge_index[0], pad_i & 8191]).reshape(NW, NCH, 1, K)
    dst = jnp.concatenate(
        [edge_index[1], N + (pad_i & (NTRASH - 1))]).reshape(NW, NCH, 1, K)
    zeros = jnp.zeros((N, D), jnp.float32)

    dstp = jnp.concatenate(
        [edge_index[1],
         jnp.arange(EPAD, dtype=jnp.int32)]).reshape(NS, NCC, 16, 128)
    etp = jnp.concatenate(
        [edge_type, jnp.full((EPAD,), R + 55, jnp.int32)]).reshape(NS, NCC, 16, 128)

    c2 = _sc_cnt(dstp, etp)
    c0 = c2[:NLOC].reshape(N, RH)
    c1 = c2[MC:MC + NLOC].reshape(N, RH)

    agg1 = _sc_agg(x, src, dst, zeros)
    h = _dense(agg1, c0, c1, x, rel1, W_I1, W_O1, b1.reshape(1, D), relu=True)
    agg2 = _sc_agg(h, src, dst, zeros)
    out = _dense(agg2, c0, c1, h, rel2, W_I2, W_O2, b2.reshape(1, D),
                 relu=False)
    return out


# speedup vs baseline: 1.0219x; 1.0219x over previous
import functools

import jax
import jax.numpy as jnp
from jax import lax
from jax.experimental import pallas as pl
from jax.experimental.pallas import tpu as pltpu
from jax.experimental.pallas import tpu_sc as plsc

N = 10000
E = 320000
D = 128
R = 200

NC = 2
NS = 16
NW = NC * NS
K = 128
NCH = 79
EPW = NCH * K
EPADA = NW * EPW - E
NTRASH = 128
NPAIR = (NCH - 1) // 2
RSTRIDE = 624
RSPAN = 640

RH = R // NC
NLOC = N * RH
MC = 1015808
CSTR = MC // NS
CW = 1024
TRASH = NLOC
KC = 2048
NCC = 10
EPAD = NS * NCC * KC - E
DSTRIDE = 62496
DSPAN = 62560
CW2 = DSPAN // 34

_mesh = plsc.VectorSubcoreMesh(core_axis_name="c", subcore_axis_name="s")


def _agg_phase(x_hbm, src_hbm, dst_hbm, zero_hbm, out_hbm,
                 s0, s1, d0, d1, xb0, xb1, agg_sh,
                 semse0, semse1, semd0, semd1, semg0, semg1, sems0, sems1):
    c = lax.axis_index("c")
    s = lax.axis_index("s")
    wid = c * NS + s

    pltpu.sync_copy(zero_hbm.at[pl.ds(s * RSTRIDE, RSPAN)],
                    agg_sh.at[pl.ds(s * RSTRIDE, RSPAN)])
    plsc.subcore_barrier()

    sb = (s0, s1)
    db = (d0, d1)
    xb = (xb0, xb1)
    semse = (semse0, semse1)
    semd = (semd0, semd1)
    semg = (semg0, semg1)
    sems = (sems0, sems1)

    def fetch_src(i, p):
        pltpu.async_copy(src_hbm.at[wid, i], sb[p], semse[p])

    def fetch_dst(i, p):
        pltpu.async_copy(dst_hbm.at[wid, i], db[p], semd[p])

    def issue_gather(p):
        pltpu.async_copy(x_hbm.at[sb[p].at[0]], xb[p], semg[p])

    def issue_scatter(p):
        pltpu.async_copy(xb[p], agg_sh.at[db[p].at[0]], sems[p], add=True)

    def drain(sem, ref):
        pltpu.make_async_copy(x_hbm.at[pl.ds(0, K)], ref, sem).wait()

    def drain_idx(semv, p, bufs):
        pltpu.make_async_copy(src_hbm.at[0, 0], bufs[p], semv[p]).wait()

    fetch_src(0, 0)
    fetch_src(1, 1)
    fetch_dst(0, 0)
    drain_idx(semse, 0, sb)
    issue_gather(0)

    def pair(t, carry):
        a = 2 * t
        drain(semg[0], xb[0])
        fetch_src(a + 2, 0)
        drain_idx(semd, 0, db)
        issue_scatter(0)

        @pl.when(t > 0)
        def _():
            drain(sems[1], xb[1])
        fetch_dst(a + 1, 1)
        drain_idx(semse, 1, sb)
        issue_gather(1)

        drain(semg[1], xb[1])

        @pl.when(t < NPAIR - 1)
        def _():
            fetch_src(a + 3, 1)
        drain_idx(semd, 1, db)
        issue_scatter(1)
        drain(sems[0], xb[0])
        fetch_dst(a + 2, 0)
        drain_idx(semse, 0, sb)
        issue_gather(0)
        return carry

    lax.fori_loop(0, NPAIR, pair, 0)
    drain(semg[0], xb[0])
    drain(sems[1], xb[1])
    drain_idx(semd, 0, db)
    issue_scatter(0)
    drain(sems[0], xb[0])

    plsc.subcore_barrier()
    pltpu.sync_copy(agg_sh.at[pl.ds(s * RSTRIDE, RSPAN)],
                    out_hbm.at[pl.ds(c * N + s * RSTRIDE, RSPAN)])


_sc_agg = pl.kernel(
    _agg_phase,
    out_type=jax.ShapeDtypeStruct((2 * N, D), jnp.float32),
    mesh=_mesh,
    scratch_types=[
        pltpu.VMEM((1, K), jnp.int32),
        pltpu.VMEM((1, K), jnp.int32),
        pltpu.VMEM((1, K), jnp.int32),
        pltpu.VMEM((1, K), jnp.int32),
        pltpu.VMEM((K, D), jnp.float32),
        pltpu.VMEM((K, D), jnp.float32),
        pltpu.VMEM_SHARED((N + NTRASH, D), jnp.float32),
        pltpu.SemaphoreType.DMA,
        pltpu.SemaphoreType.DMA,
        pltpu.SemaphoreType.DMA,
        pltpu.SemaphoreType.DMA,
        pltpu.SemaphoreType.DMA,
        pltpu.SemaphoreType.DMA,
        pltpu.SemaphoreType.DMA,
        pltpu.SemaphoreType.DMA,
    ],
)


def _cnt_phase(dstc_hbm, etc_hbm, outc0_hbm, outc1_hbm,
                 dstc_v, etc_v, fb, ones_v, stage_v, staged0_v, staged1_v,
                 c_sh, semsc, semz, semst):
    c = lax.axis_index("c")
    s = lax.axis_index("s")

    for l in range(CW // 16):
        stage_v[pl.ds(l * 16, 16)] = jnp.full((16,), 0.0, jnp.float32)

    def zbody(j, carry):
        pltpu.async_copy(stage_v, c_sh.at[pl.ds(s * CSTR + j * CW, CW)], semz)
        return carry

    lax.fori_loop(0, CSTR // CW, zbody, 0)
    pltpu.sync_copy(dstc_hbm.at[s], dstc_v)
    pltpu.sync_copy(etc_hbm.at[s], etc_v)
    for l in range(8):
        ones_v[0, pl.ds(l * 16, 16)] = jnp.full((16,), 1.0, jnp.float32)

    def zdrain(j, carry):
        pltpu.make_async_copy(outc0_hbm.at[pl.ds(0, CW)], stage_v, semz).wait()
        return carry

    lax.fori_loop(0, CSTR // CW, zdrain, 0)
    plsc.subcore_barrier()

    lo = c * RH

    def cbody(i, carry):
        for r in range(16):
            for l in range(8):
                dv = dstc_v[i, r, pl.ds(l * 16, 16)]
                ev = etc_v[i, r, pl.ds(l * 16, 16)]
                el = ev - lo
                ok = (el >= 0) & (el < RH)
                f = jnp.where(ok, dv * RH + el, TRASH + (dv & 4095))
                fb[i, r, 0, pl.ds(l * 16, 16)] = f
        for r in range(16):
            pltpu.async_copy(ones_v.at[0], c_sh.at[fb.at[i, r, 0]],
                             semsc, add=True)
        return carry

    lax.fori_loop(0, NCC, cbody, 0)

    def dr(i, carry):
        pltpu.make_async_copy(dstc_hbm.at[0, 0], dstc_v.at[0], semsc).wait()
        return carry

    lax.fori_loop(0, NCC, dr, 0)
    plsc.subcore_barrier()

    def dump_to(out_hbm):
        def dpair(u, carry):
            j0 = s * DSTRIDE + 2 * u * CW2
            j1 = j0 + CW2

            @pl.when(u > 0)
            def _():
                pltpu.make_async_copy(out_hbm.at[pl.ds(0, CW2)], staged0_v,
                                      semst).wait()
                pltpu.make_async_copy(out_hbm.at[pl.ds(0, CW2)], staged1_v,
                                      semst).wait()
            pltpu.sync_copy(c_sh.at[pl.ds(j0, CW2)], staged0_v)
            pltpu.async_copy(staged0_v, out_hbm.at[pl.ds(j0, CW2)], semst)
            pltpu.sync_copy(c_sh.at[pl.ds(j1, CW2)], staged1_v)
            pltpu.async_copy(staged1_v, out_hbm.at[pl.ds(j1, CW2)], semst)
            return carry

        lax.fori_loop(0, DSPAN // CW2 // 2, dpair, 0)
        pltpu.make_async_copy(out_hbm.at[pl.ds(0, CW2)], staged0_v,
                              semst).wait()
        pltpu.make_async_copy(out_hbm.at[pl.ds(0, CW2)], staged1_v,
                              semst).wait()

    @pl.when(c == 0)
    def _():
        dump_to(outc0_hbm)

    @pl.when(c == 1)
    def _():
        dump_to(outc1_hbm)


_sc_cnt = pl.kernel(
    _cnt_phase,
    out_type=(jax.ShapeDtypeStruct((NLOC,), jnp.float32),
              jax.ShapeDtypeStruct((NLOC,), jnp.float32)),
    mesh=_mesh,
    scratch_types=[
        pltpu.VMEM((NCC, 16, 128), jnp.int32),
        pltpu.VMEM((NCC, 16, 128), jnp.int32),
        pltpu.VMEM((NCC, 16, 1, 128), jnp.int32),
        pltpu.VMEM((1, 128), jnp.float32),
        pltpu.VMEM((CW,), jnp.float32),
        pltpu.VMEM((CW2,), jnp.float32),
        pltpu.VMEM((CW2,), jnp.float32),
        pltpu.VMEM_SHARED((MC,), jnp.float32),
        pltpu.SemaphoreType.DMA,
        pltpu.SemaphoreType.DMA,
        pltpu.SemaphoreType.DMA,
    ],
)


def _dense_body(relu, a0_ref, a1_ref, x_ref, c0_ref, c1_ref, rel_ref, wi_ref,
                wo_ref, b_ref, o_ref):
    xb = x_ref[...]
    m = a0_ref[...] + a1_ref[...] + xb
    dn = (((1,), (1,)), ((), ()))
    dc = (((1,), (0,)), ((), ()))
    wi = wi_ref[...]
    relw = lax.dot_general(rel_ref[...], wi, dn,
                           preferred_element_type=jnp.float32)
    t = lax.dot_general(m, wi, dn, preferred_element_type=jnp.float32)
    t = t + lax.dot_general(xb, wo_ref[...], dn,
                            preferred_element_type=jnp.float32)
    t = t - lax.dot_general(c0_ref[...], relw[0:RH, :], dc,
                            preferred_element_type=jnp.float32)
    t = t - lax.dot_general(c1_ref[...], relw[RH:R, :], dc,
                            preferred_element_type=jnp.float32)
    t = t + (b_ref[...] - relw[0:1, :])
    o_ref[...] = jnp.maximum(t, 0.0) if relu else t


BLK = 2000
GRID = N // BLK


def _dense(agg2, c0, c1, x, rel, w_i, w_o, b, relu):
    return pl.pallas_call(
        functools.partial(_dense_body, relu),
        grid=(GRID,),
        in_specs=[
            pl.BlockSpec((BLK, D), lambda i: (i, 0)),
            pl.BlockSpec((BLK, D), lambda i: (i + GRID, 0)),
            pl.BlockSpec((BLK, D), lambda i: (i, 0)),
            pl.BlockSpec((BLK, RH), lambda i: (i, 0)),
            pl.BlockSpec((BLK, RH), lambda i: (i, 0)),
            pl.BlockSpec((R, D), lambda i: (0, 0)),
            pl.BlockSpec((D, D), lambda i: (0, 0)),
            pl.BlockSpec((D, D), lambda i: (0, 0)),
            pl.BlockSpec((1, D), lambda i: (0, 0)),
        ],
        out_specs=pl.BlockSpec((BLK, D), lambda i: (i, 0)),
        out_shape=jax.ShapeDtypeStruct((N, D), jnp.float32),
    )(agg2, agg2, x, c0, c1, rel, w_i, w_o, b)


def kernel(x, edge_index, edge_type, W_I1, W_O1, W_R1, rel1, b1,
           W_I2, W_O2, W_R2, rel2, b2):
    pad_i = jnp.arange(EPADA, dtype=jnp.int32)
    src = jnp.concatenate(
        [edge_index[0], pad_i & 8191]).reshape(NW, NCH, 1, K)
    dst = jnp.concatenate(
        [edge_index[1], N + (pad_i & (NTRASH - 1))]).reshape(NW, NCH, 1, K)
    zeros = jnp.zeros((N, D), jnp.float32)

    dstp = jnp.concatenate(
        [edge_index[1],
         jnp.arange(EPAD, dtype=jnp.int32)]).reshape(NS, NCC, 16, 128)
    etp = jnp.concatenate(
        [edge_type, jnp.full((EPAD,), R + 55, jnp.int32)]).reshape(NS, NCC, 16, 128)

    c0f, c1f = _sc_cnt(dstp, etp)
    c0 = c0f.reshape(N, RH)
    c1 = c1f.reshape(N, RH)

    agg1 = _sc_agg(x, src, dst, zeros)
    h = _dense(agg1, c0, c1, x, rel1, W_I1, W_O1, b1.reshape(1, D), relu=True)
    agg2 = _sc_agg(h, src, dst, zeros)
    out = _dense(agg2, c0, c1, h, rel2, W_I2, W_O2, b2.reshape(1, D),
                 relu=False)
    return out
